# baseline (speedup 1.0000x reference)
import functools

import jax
import jax.numpy as jnp
from jax import lax
from jax.experimental import pallas as pl
from jax.experimental.pallas import tpu as pltpu
from jax.experimental.pallas import tpu_sc as plsc

D = 256
EPS = 1e-8
EP = 40960
NTILES = 16
CH = 64
NCHUNK = EP // (NTILES * CH)


def _ht_body(x_ref, w_ref, b_ref, ht_ref, vh_ref):
    ht = lax.dot_general(x_ref[...], w_ref[...], (((1,), (1,)), ((), ())),
                         preferred_element_type=jnp.float32) + b_ref[...]
    n = jnp.maximum(jnp.sqrt(jnp.sum(ht * ht, axis=1, keepdims=True)), EPS)
    ht_ref[...] = ht
    vh_ref[...] = ht / n


def _ht_vh(x, w, b, bm=1000):
    m = x.shape[0]
    return pl.pallas_call(
        _ht_body,
        grid=(m // bm,),
        in_specs=[pl.BlockSpec((bm, D), lambda i: (i, 0)),
                  pl.BlockSpec((D, D), lambda i: (0, 0)),
                  pl.BlockSpec((1, D), lambda i: (0, 0))],
        out_specs=[pl.BlockSpec((bm, D), lambda i: (i, 0)),
                   pl.BlockSpec((bm, D), lambda i: (i, 0))],
        out_shape=[jax.ShapeDtypeStruct((m, D), jnp.float32),
                   jax.ShapeDtypeStruct((m, D), jnp.float32)],
    )(x, w, b.reshape(1, D))


def _q_body(x_ref, w_ref, o_ref, *, c):
    q0 = lax.dot_general(x_ref[...], w_ref[...], (((1,), (1,)), ((), ())),
                         preferred_element_type=jnp.float32)
    n = jnp.maximum(jnp.sqrt(jnp.sum(q0 * q0, axis=1, keepdims=True)), EPS)
    o_ref[...] = q0 * jnp.sqrt(c / n)


def _q(x, w, c, bm=1000):
    m = x.shape[0]
    return pl.pallas_call(
        functools.partial(_q_body, c=c),
        grid=(m // bm,),
        in_specs=[pl.BlockSpec((bm, D), lambda i: (i, 0)),
                  pl.BlockSpec((D, D), lambda i: (0, 0))],
        out_specs=pl.BlockSpec((bm, D), lambda i: (i, 0)),
        out_shape=jax.ShapeDtypeStruct((m, D), jnp.float32),
    )(x, w)


def _cat_body(ht_ref, a_ref, b_ref, o_ref):
    o_ref[:, :D] = ht_ref[...]
    o_ref[:, D:] = a_ref[...] + b_ref[...]


def _cat_add(ht, a, b, bm=1000):
    m = ht.shape[0]
    return pl.pallas_call(
        _cat_body,
        grid=(m // bm,),
        in_specs=[pl.BlockSpec((bm, D), lambda i: (i, 0))] * 3,
        out_specs=pl.BlockSpec((bm, 2 * D), lambda i: (i, 0)),
        out_shape=jax.ShapeDtypeStruct((m, 2 * D), jnp.float32),
    )(ht, a, b)


def _edge_kernel(q0_tab, vh0_tab, s0_h, d0_h, q1_tab, vh1_tab, s1_h, d1_h,
                 zrows,
                 m0_ref, m1_ref,
                 acc, sidx, didx, qrows, vrows, semq, semv,
                 *, acc_rows, n_out):
    cid = lax.axis_index("c")
    sid = lax.axis_index("s")

    zcopies = acc_rows // (NTILES * CH)
    pltpu.sync_copy(zrows, qrows)
    for k in range(zcopies):
        pltpu.sync_copy(qrows, acc.at[pl.ds(sid * (zcopies * CH) + k * CH,
                                            CH)])
    plsc.subcore_barrier()

    rows0 = [lax.iota(jnp.int32, 16) + g * 16 for g in range(CH // 16)]

    def do_rel(q_tab, vh_tab, sidx_h, didx_h):
        pltpu.sync_copy(sidx_h.at[sid], sidx)
        pltpu.sync_copy(didx_h.at[sid], didx)

        def chunk(j, carry):
            cq = pltpu.async_copy(q_tab.at[sidx.at[j]], qrows, semq)
            cv = pltpu.async_copy(vh_tab.at[didx.at[j]], vrows, semv)
            cq.wait()
            cv.wait()

            @plsc.parallel_loop(0, CH, step=1, unroll=2)
            def _(e):
                qs = []
                t = None
                for k in range(D // 16):
                    qk = qrows[e, pl.ds(k * 16, 16)]
                    vk = vrows[e, pl.ds(k * 16, 16)]
                    qs.append(qk)
                    t = qk * vk if t is None else t + qk * vk
                s = jnp.sum(t)
                for k in range(D // 16):
                    qrows[e, pl.ds(k * 16, 16)] = qs[k] * s

            pltpu.sync_copy(qrows, acc.at[didx.at[j]], add=True)
            return carry

        lax.fori_loop(0, NCHUNK, chunk, jnp.int32(0))

    @pl.when(cid == 0)
    def _():
        do_rel(q0_tab, vh0_tab, s0_h, d0_h)

    @pl.when(cid == 1)
    def _():
        do_rel(q1_tab, vh1_tab, s1_h, d1_h)

    plsc.subcore_barrier()

    main = (n_out // (NTILES * 8)) * 8
    last = n_out - 15 * main
    out_ref = [m0_ref, m1_ref]
    for c in range(2):
        @pl.when(cid == c)
        def _(oref=out_ref[c]):
            @pl.when(sid < 15)
            def _():
                pltpu.sync_copy(acc.at[pl.ds(sid * main, main)],
                                oref.at[pl.ds(sid * main, main)])

            @pl.when(sid == 15)
            def _():
                pltpu.sync_copy(acc.at[pl.ds(15 * main, last)],
                                oref.at[pl.ds(15 * main, last)])


def _sc_pass(q0, vh0, s0, d0, q1, vh1, s1, d1, n_out):
    acc_rows = ((n_out + NTILES * CH) // (NTILES * CH)) * (NTILES * CH)
    mesh = plsc.VectorSubcoreMesh(core_axis_name="c", subcore_axis_name="s")
    f = pl.kernel(
        functools.partial(_edge_kernel, acc_rows=acc_rows, n_out=n_out),
        mesh=mesh,
        compiler_params=pltpu.CompilerParams(use_tc_tiling_on_sc=False,
                                             needs_layout_passes=False),
        out_type=[jax.ShapeDtypeStruct((n_out, D), jnp.float32),
                  jax.ShapeDtypeStruct((n_out, D), jnp.float32)],
        scratch_types=[
            pltpu.VMEM_SHARED((acc_rows, D), jnp.float32),
            pltpu.VMEM((NCHUNK, CH), jnp.int32),
            pltpu.VMEM((NCHUNK, CH), jnp.int32),
            pltpu.VMEM((CH, D), jnp.float32),
            pltpu.VMEM((CH, D), jnp.float32),
            pltpu.SemaphoreType.DMA,
            pltpu.SemaphoreType.DMA,
        ],
    )
    zrows = jnp.zeros((CH, D), jnp.float32)
    return f(q0, vh0, s0, d0, q1, vh1, s1, d1, zrows)


def _prep_idx(idx, pad_val):
    e = idx.shape[0]
    a = jnp.concatenate([idx.astype(jnp.int32),
                         jnp.full((EP - e,), pad_val, jnp.int32)])
    return a.reshape(NTILES, NCHUNK, CH)


def _pad_rows(x, n):
    return jnp.zeros((n, D), jnp.float32).at[:x.shape[0]].set(x)


def kernel(feat_vul, feat_weakness_name, feat_other, src_w2v, dst_w2v,
           src_o2v, dst_o2v, src_v2w, dst_v2w, src_v2o, dst_v2o,
           W_w2v, W_o2v, W_v2w, W_v2o,
           Wn_vul, bn_vul, Wn_weakness_name, bn_weakness_name,
           Wn_other, bn_other):
    nv, nw, no = feat_vul.shape[0], feat_weakness_name.shape[0], feat_other.shape[0]
    ht_vul, vh_vul = _ht_vh(feat_vul, Wn_vul, bn_vul)
    ht_w, vh_w = _ht_vh(feat_weakness_name, Wn_weakness_name,
                        bn_weakness_name)
    ht_o, vh_o = _ht_vh(feat_other, Wn_other, bn_other)

    q_w2v = _q(feat_weakness_name, W_w2v, 0.6)
    q_o2v = _q(feat_other, W_o2v, 0.4)
    q_v2w = _q(feat_vul, W_v2w, 1.0)
    q_v2o = _q(feat_vul, W_v2o, 1.0)

    vh_vul_p = _pad_rows(vh_vul, nv + 8)
    vh_w_p = _pad_rows(vh_w, nw + 8)
    vh_o_p = _pad_rows(vh_o, no + 8)

    mv0, mv1 = _sc_pass(
        q_w2v, vh_vul_p, _prep_idx(src_w2v, 0), _prep_idx(dst_w2v, nv),
        q_o2v, vh_vul_p, _prep_idx(src_o2v, 0), _prep_idx(dst_o2v, nv),
        nv)
    mw, mo = _sc_pass(
        q_v2w, vh_w_p, _prep_idx(src_v2w, 0), _prep_idx(dst_v2w, nw),
        q_v2o, vh_o_p, _prep_idx(src_v2o, 0), _prep_idx(dst_v2o, no),
        nw)

    zero_w = jnp.zeros((nw, D), jnp.float32)
    out_vul = _cat_add(ht_vul, mv0, mv1)
    out_w = _cat_add(ht_w, mw, zero_w)
    out_o = _cat_add(ht_o, mo, zero_w)
    return (out_vul, out_w, out_o)


# speedup vs baseline: 1.2857x; 1.2857x over previous
import functools

import jax
import jax.numpy as jnp
from jax import lax
from jax.experimental import pallas as pl
from jax.experimental.pallas import tpu as pltpu
from jax.experimental.pallas import tpu_sc as plsc

D = 256
EPS = 1e-8
EP = 40960
NTILES = 16
CH = 32
NCHUNK = EP // (NTILES * CH)


def _ht_body(x_ref, w_ref, b_ref, ht_ref, vh_ref):
    ht = lax.dot_general(x_ref[...], w_ref[...], (((1,), (1,)), ((), ())),
                         preferred_element_type=jnp.float32) + b_ref[...]
    n = jnp.maximum(jnp.sqrt(jnp.sum(ht * ht, axis=1, keepdims=True)), EPS)
    ht_ref[...] = ht
    vh_ref[...] = ht / n


def _ht_vh(x, w, b, bm=1000):
    m = x.shape[0]
    return pl.pallas_call(
        _ht_body,
        grid=(m // bm,),
        in_specs=[pl.BlockSpec((bm, D), lambda i: (i, 0)),
                  pl.BlockSpec((D, D), lambda i: (0, 0)),
                  pl.BlockSpec((1, D), lambda i: (0, 0))],
        out_specs=[pl.BlockSpec((bm, D), lambda i: (i, 0)),
                   pl.BlockSpec((bm, D), lambda i: (i, 0))],
        out_shape=[jax.ShapeDtypeStruct((m, D), jnp.float32),
                   jax.ShapeDtypeStruct((m, D), jnp.float32)],
    )(x, w, b.reshape(1, D))


def _q_body(x_ref, w_ref, o_ref, *, c):
    q0 = lax.dot_general(x_ref[...], w_ref[...], (((1,), (1,)), ((), ())),
                         preferred_element_type=jnp.float32)
    n = jnp.maximum(jnp.sqrt(jnp.sum(q0 * q0, axis=1, keepdims=True)), EPS)
    o_ref[...] = q0 * jnp.sqrt(c / n)


def _q(x, w, c, bm=1000):
    m = x.shape[0]
    return pl.pallas_call(
        functools.partial(_q_body, c=c),
        grid=(m // bm,),
        in_specs=[pl.BlockSpec((bm, D), lambda i: (i, 0)),
                  pl.BlockSpec((D, D), lambda i: (0, 0))],
        out_specs=pl.BlockSpec((bm, D), lambda i: (i, 0)),
        out_shape=jax.ShapeDtypeStruct((m, D), jnp.float32),
    )(x, w)


def _cat_body(ht_ref, a_ref, b_ref, o_ref):
    o_ref[:, :D] = ht_ref[...]
    o_ref[:, D:] = a_ref[...] + b_ref[...]


def _cat_add(ht, a, b, bm=1000):
    m = ht.shape[0]
    return pl.pallas_call(
        _cat_body,
        grid=(m // bm,),
        in_specs=[pl.BlockSpec((bm, D), lambda i: (i, 0))] * 3,
        out_specs=pl.BlockSpec((bm, 2 * D), lambda i: (i, 0)),
        out_shape=jax.ShapeDtypeStruct((m, 2 * D), jnp.float32),
    )(ht, a, b)


def _edge_kernel(q0_tab, vh0_tab, s0_h, d0_h, q1_tab, vh1_tab, s1_h, d1_h,
                 zrows,
                 m0_ref, m1_ref,
                 acc, sidx, didx, qrows0, qrows1, vrows0, vrows1,
                 semq0, semq1, semv0, semv1,
                 *, acc_rows, n_out):
    cid = lax.axis_index("c")
    sid = lax.axis_index("s")
    qrows = [qrows0, qrows1]
    vrows = [vrows0, vrows1]
    semq = [semq0, semq1]
    semv = [semv0, semv1]

    zcopies = acc_rows // (NTILES * CH)
    pltpu.sync_copy(zrows, qrows0)
    for k in range(zcopies):
        pltpu.sync_copy(qrows0, acc.at[pl.ds(sid * (zcopies * CH) + k * CH,
                                             CH)])
    plsc.subcore_barrier()

    def do_rel(q_tab, vh_tab, sidx_h, didx_h):
        pltpu.sync_copy(sidx_h.at[sid], sidx)
        pltpu.sync_copy(didx_h.at[sid], didx)

        def fire(jj, b):
            pltpu.async_copy(q_tab.at[sidx.at[jj]], qrows[b], semq[b])
            pltpu.async_copy(vh_tab.at[didx.at[jj]], vrows[b], semv[b])

        def drain_compute_scatter(jj, b):
            pltpu.make_async_copy(q_tab.at[sidx.at[jj]], qrows[b],
                                  semq[b]).wait()
            pltpu.make_async_copy(vh_tab.at[didx.at[jj]], vrows[b],
                                  semv[b]).wait()

            @plsc.parallel_loop(0, CH, step=1, unroll=2)
            def _(e):
                qs = []
                t = None
                for k in range(D // 16):
                    qk = qrows[b][e, pl.ds(k * 16, 16)]
                    vk = vrows[b][e, pl.ds(k * 16, 16)]
                    qs.append(qk)
                    t = qk * vk if t is None else t + qk * vk
                s = jnp.sum(t)
                for k in range(D // 16):
                    qrows[b][e, pl.ds(k * 16, 16)] = qs[k] * s

            pltpu.sync_copy(qrows[b], acc.at[didx.at[jj]], add=True)

        fire(jnp.int32(0), 0)
        fire(jnp.int32(1), 1)

        def chunk2(j, carry):
            for b in range(2):
                jj = j + b
                drain_compute_scatter(jj, b)

                @pl.when(jj + 2 < NCHUNK)
                def _():
                    fire(jj + 2, b)
            return carry

        lax.fori_loop(0, NCHUNK // 2, lambda i, c: chunk2(i * 2, c),
                      jnp.int32(0))

    @pl.when(cid == 0)
    def _():
        do_rel(q0_tab, vh0_tab, s0_h, d0_h)

    @pl.when(cid == 1)
    def _():
        do_rel(q1_tab, vh1_tab, s1_h, d1_h)

    plsc.subcore_barrier()

    main = (n_out // (NTILES * 8)) * 8
    last = n_out - 15 * main
    out_ref = [m0_ref, m1_ref]
    for c in range(2):
        @pl.when(cid == c)
        def _(oref=out_ref[c]):
            @pl.when(sid < 15)
            def _():
                pltpu.sync_copy(acc.at[pl.ds(sid * main, main)],
                                oref.at[pl.ds(sid * main, main)])

            @pl.when(sid == 15)
            def _():
                pltpu.sync_copy(acc.at[pl.ds(15 * main, last)],
                                oref.at[pl.ds(15 * main, last)])


def _sc_pass(q0, vh0, s0, d0, q1, vh1, s1, d1, n_out):
    acc_rows = ((n_out + NTILES * CH) // (NTILES * CH)) * (NTILES * CH)
    mesh = plsc.VectorSubcoreMesh(core_axis_name="c", subcore_axis_name="s")
    f = pl.kernel(
        functools.partial(_edge_kernel, acc_rows=acc_rows, n_out=n_out),
        mesh=mesh,
        compiler_params=pltpu.CompilerParams(use_tc_tiling_on_sc=False,
                                             needs_layout_passes=False),
        out_type=[jax.ShapeDtypeStruct((n_out, D), jnp.float32),
                  jax.ShapeDtypeStruct((n_out, D), jnp.float32)],
        scratch_types=[
            pltpu.VMEM_SHARED((acc_rows, D), jnp.float32),
            pltpu.VMEM((NCHUNK, CH), jnp.int32),
            pltpu.VMEM((NCHUNK, CH), jnp.int32),
            pltpu.VMEM((CH, D), jnp.float32),
            pltpu.VMEM((CH, D), jnp.float32),
            pltpu.VMEM((CH, D), jnp.float32),
            pltpu.VMEM((CH, D), jnp.float32),
            pltpu.SemaphoreType.DMA,
            pltpu.SemaphoreType.DMA,
            pltpu.SemaphoreType.DMA,
            pltpu.SemaphoreType.DMA,
        ],
    )
    zrows = jnp.zeros((CH, D), jnp.float32)
    return f(q0, vh0, s0, d0, q1, vh1, s1, d1, zrows)


def _prep_idx(idx, pad_val):
    e = idx.shape[0]
    a = jnp.concatenate([idx.astype(jnp.int32),
                         jnp.full((EP - e,), pad_val, jnp.int32)])
    return a.reshape(NTILES, NCHUNK, CH)


def _pad_rows(x, n):
    return jnp.zeros((n, D), jnp.float32).at[:x.shape[0]].set(x)


def kernel(feat_vul, feat_weakness_name, feat_other, src_w2v, dst_w2v,
           src_o2v, dst_o2v, src_v2w, dst_v2w, src_v2o, dst_v2o,
           W_w2v, W_o2v, W_v2w, W_v2o,
           Wn_vul, bn_vul, Wn_weakness_name, bn_weakness_name,
           Wn_other, bn_other):
    nv, nw, no = feat_vul.shape[0], feat_weakness_name.shape[0], feat_other.shape[0]
    ht_vul, vh_vul = _ht_vh(feat_vul, Wn_vul, bn_vul)
    ht_w, vh_w = _ht_vh(feat_weakness_name, Wn_weakness_name,
                        bn_weakness_name)
    ht_o, vh_o = _ht_vh(feat_other, Wn_other, bn_other)

    q_w2v = _q(feat_weakness_name, W_w2v, 0.6)
    q_o2v = _q(feat_other, W_o2v, 0.4)
    q_v2w = _q(feat_vul, W_v2w, 1.0)
    q_v2o = _q(feat_vul, W_v2o, 1.0)

    vh_vul_p = _pad_rows(vh_vul, nv + 8)
    vh_w_p = _pad_rows(vh_w, nw + 8)
    vh_o_p = _pad_rows(vh_o, no + 8)

    mv0, mv1 = _sc_pass(
        q_w2v, vh_vul_p, _prep_idx(src_w2v, 0), _prep_idx(dst_w2v, nv),
        q_o2v, vh_vul_p, _prep_idx(src_o2v, 0), _prep_idx(dst_o2v, nv),
        nv)
    mw, mo = _sc_pass(
        q_v2w, vh_w_p, _prep_idx(src_v2w, 0), _prep_idx(dst_v2w, nw),
        q_v2o, vh_o_p, _prep_idx(src_v2o, 0), _prep_idx(dst_v2o, no),
        nw)

    zero_w = jnp.zeros((nw, D), jnp.float32)
    out_vul = _cat_add(ht_vul, mv0, mv1)
    out_w = _cat_add(ht_w, mw, zero_w)
    out_o = _cat_add(ht_o, mo, zero_w)
    return (out_vul, out_w, out_o)


# speedup vs baseline: 1.4443x; 1.1234x over previous
import functools

import jax
import jax.numpy as jnp
from jax import lax
from jax.experimental import pallas as pl
from jax.experimental.pallas import tpu as pltpu
from jax.experimental.pallas import tpu_sc as plsc

D = 256
EPS = 1e-8
EP = 40960
NTILES = 16
CH = 32
NCHUNK = EP // (NTILES * CH)


def _ht_body(x_ref, w_ref, b_ref, ht_ref, vh_ref):
    ht = lax.dot_general(x_ref[...], w_ref[...], (((1,), (1,)), ((), ())),
                         preferred_element_type=jnp.float32) + b_ref[...]
    n = jnp.maximum(jnp.sqrt(jnp.sum(ht * ht, axis=1, keepdims=True)), EPS)
    ht_ref[...] = ht
    vh_ref[...] = ht / n


def _ht_vh(x, w, b, bm=1000):
    m = x.shape[0]
    return pl.pallas_call(
        _ht_body,
        grid=(m // bm,),
        in_specs=[pl.BlockSpec((bm, D), lambda i: (i, 0)),
                  pl.BlockSpec((D, D), lambda i: (0, 0)),
                  pl.BlockSpec((1, D), lambda i: (0, 0))],
        out_specs=[pl.BlockSpec((bm, D), lambda i: (i, 0)),
                   pl.BlockSpec((bm, D), lambda i: (i, 0))],
        out_shape=[jax.ShapeDtypeStruct((m, D), jnp.float32),
                   jax.ShapeDtypeStruct((m, D), jnp.float32)],
    )(x, w, b.reshape(1, D))


def _q_body(x_ref, w_ref, o_ref, *, c):
    q0 = lax.dot_general(x_ref[...], w_ref[...], (((1,), (1,)), ((), ())),
                         preferred_element_type=jnp.float32)
    n = jnp.maximum(jnp.sqrt(jnp.sum(q0 * q0, axis=1, keepdims=True)), EPS)
    o_ref[...] = q0 * jnp.sqrt(c / n)


def _q(x, w, c, bm=1000):
    m = x.shape[0]
    return pl.pallas_call(
        functools.partial(_q_body, c=c),
        grid=(m // bm,),
        in_specs=[pl.BlockSpec((bm, D), lambda i: (i, 0)),
                  pl.BlockSpec((D, D), lambda i: (0, 0))],
        out_specs=pl.BlockSpec((bm, D), lambda i: (i, 0)),
        out_shape=jax.ShapeDtypeStruct((m, D), jnp.float32),
    )(x, w)


def _cat_body2(ht_ref, a_ref, o_ref):
    o_ref[:, :D] = ht_ref[...]
    o_ref[:, D:] = a_ref[...]


def _cat_body3(ht_ref, a_ref, b_ref, o_ref):
    o_ref[:, :D] = ht_ref[...]
    o_ref[:, D:] = a_ref[...] + b_ref[...]


def _cat_add(ht, a, b=None, bm=1000):
    m = ht.shape[0]
    args = (ht, a) if b is None else (ht, a, b)
    return pl.pallas_call(
        _cat_body2 if b is None else _cat_body3,
        grid=(m // bm,),
        in_specs=[pl.BlockSpec((bm, D), lambda i: (i, 0))] * len(args),
        out_specs=pl.BlockSpec((bm, 2 * D), lambda i: (i, 0)),
        out_shape=jax.ShapeDtypeStruct((m, 2 * D), jnp.float32),
    )(*args)


def _pack_vh(vh_p):
    n = vh_p.shape[0]
    u = lax.bitcast_convert_type(vh_p.astype(jnp.bfloat16), jnp.uint16)
    ur = u.reshape(n, D // 32, 2, 16).astype(jnp.uint32)
    w = ur[:, :, 0, :] | (ur[:, :, 1, :] << 16)
    return lax.bitcast_convert_type(w, jnp.int32).reshape(n, D // 2)


NBUF = 4


def _edge_kernel(q0_tab, vh0_tab, s0_h, d0_h, q1_tab, vh1_tab, s1_h, d1_h,
                 zrows,
                 m0_ref, m1_ref,
                 acc, sidx, didx, *bufs_and_sems,
                 **kw):
    acc_rows = kw["acc_rows"]
    n_out = kw["n_out"]
    qrows = list(bufs_and_sems[0:NBUF])
    vrows = list(bufs_and_sems[NBUF:2 * NBUF])
    semq = list(bufs_and_sems[2 * NBUF:3 * NBUF])
    semv = list(bufs_and_sems[3 * NBUF:4 * NBUF])
    cid = lax.axis_index("c")
    sid = lax.axis_index("s")
    qrows0 = qrows[0]

    zcopies = acc_rows // (NTILES * CH)
    pltpu.sync_copy(zrows, qrows0)
    for k in range(zcopies):
        pltpu.sync_copy(qrows0, acc.at[pl.ds(sid * (zcopies * CH) + k * CH,
                                             CH)])
    plsc.subcore_barrier()

    def do_rel(q_tab, vh_tab, sidx_h, didx_h):
        pltpu.sync_copy(sidx_h.at[sid], sidx)
        pltpu.sync_copy(didx_h.at[sid], didx)

        def fire(jj, b):
            pltpu.async_copy(q_tab.at[sidx.at[jj]], qrows[b], semq[b])
            pltpu.async_copy(vh_tab.at[didx.at[jj]], vrows[b], semv[b])

        def drain_compute_scatter(jj, b):
            pltpu.make_async_copy(q_tab.at[sidx.at[jj]], qrows[b],
                                  semq[b]).wait()
            pltpu.make_async_copy(vh_tab.at[didx.at[jj]], vrows[b],
                                  semv[b]).wait()
            hi_mask = jnp.full((16,), -65536, jnp.int32)

            @plsc.parallel_loop(0, CH, step=1, unroll=2)
            def _(e):
                qs = []
                t = None
                for j in range(D // 32):
                    vw = vrows[b][e, pl.ds(j * 16, 16)]
                    vlo = plsc.bitcast(vw << 16, jnp.float32)
                    vhi = plsc.bitcast(vw & hi_mask, jnp.float32)
                    qa = qrows[b][e, pl.ds(j * 32, 16)]
                    qb = qrows[b][e, pl.ds(j * 32 + 16, 16)]
                    qs.append(qa)
                    qs.append(qb)
                    p = qa * vlo + qb * vhi
                    t = p if t is None else t + p
                s = jnp.sum(t)
                for k in range(D // 16):
                    qrows[b][e, pl.ds(k * 16, 16)] = qs[k] * s

            pltpu.sync_copy(qrows[b], acc.at[didx.at[jj]], add=True)

        for b in range(NBUF):
            fire(jnp.int32(b), b)

        def chunkn(j, carry):
            for b in range(NBUF):
                jj = j + b
                drain_compute_scatter(jj, b)

                @pl.when(jj + NBUF < NCHUNK)
                def _():
                    fire(jj + NBUF, b)
            return carry

        lax.fori_loop(0, NCHUNK // NBUF, lambda i, c: chunkn(i * NBUF, c),
                      jnp.int32(0))

    @pl.when(cid == 0)
    def _():
        do_rel(q0_tab, vh0_tab, s0_h, d0_h)

    @pl.when(cid == 1)
    def _():
        do_rel(q1_tab, vh1_tab, s1_h, d1_h)

    plsc.subcore_barrier()

    main = (n_out // (NTILES * 8)) * 8
    last = n_out - 15 * main
    out_ref = [m0_ref, m1_ref]
    for c in range(2):
        @pl.when(cid == c)
        def _(oref=out_ref[c]):
            @pl.when(sid < 15)
            def _():
                pltpu.sync_copy(acc.at[pl.ds(sid * main, main)],
                                oref.at[pl.ds(sid * main, main)])

            @pl.when(sid == 15)
            def _():
                pltpu.sync_copy(acc.at[pl.ds(15 * main, last)],
                                oref.at[pl.ds(15 * main, last)])


def _sc_pass(q0, vh0, s0, d0, q1, vh1, s1, d1, n_out):
    acc_rows = ((n_out + NTILES * CH) // (NTILES * CH)) * (NTILES * CH)
    mesh = plsc.VectorSubcoreMesh(core_axis_name="c", subcore_axis_name="s")
    f = pl.kernel(
        functools.partial(_edge_kernel, acc_rows=acc_rows, n_out=n_out),
        mesh=mesh,
        compiler_params=pltpu.CompilerParams(use_tc_tiling_on_sc=False,
                                             needs_layout_passes=False),
        out_type=[jax.ShapeDtypeStruct((n_out, D), jnp.float32),
                  jax.ShapeDtypeStruct((n_out, D), jnp.float32)],
        scratch_types=([
            pltpu.VMEM_SHARED((acc_rows, D), jnp.float32),
            pltpu.VMEM((NCHUNK, CH), jnp.int32),
            pltpu.VMEM((NCHUNK, CH), jnp.int32),
        ] + [pltpu.VMEM((CH, D), jnp.float32)] * NBUF
            + [pltpu.VMEM((CH, D // 2), jnp.int32)] * NBUF
            + [pltpu.SemaphoreType.DMA] * (2 * NBUF)),
    )
    zrows = jnp.zeros((CH, D), jnp.float32)
    return f(q0, vh0, s0, d0, q1, vh1, s1, d1, zrows)


def _prep_idx(idx, pad_val):
    e = idx.shape[0]
    a = jnp.concatenate([idx.astype(jnp.int32),
                         jnp.full((EP - e,), pad_val, jnp.int32)])
    return a.reshape(NTILES, NCHUNK, CH)


def _pad_rows(x, n):
    return jnp.zeros((n, D), jnp.float32).at[:x.shape[0]].set(x)


def kernel(feat_vul, feat_weakness_name, feat_other, src_w2v, dst_w2v,
           src_o2v, dst_o2v, src_v2w, dst_v2w, src_v2o, dst_v2o,
           W_w2v, W_o2v, W_v2w, W_v2o,
           Wn_vul, bn_vul, Wn_weakness_name, bn_weakness_name,
           Wn_other, bn_other):
    nv, nw, no = feat_vul.shape[0], feat_weakness_name.shape[0], feat_other.shape[0]
    ht_vul, vh_vul = _ht_vh(feat_vul, Wn_vul, bn_vul)
    ht_w, vh_w = _ht_vh(feat_weakness_name, Wn_weakness_name,
                        bn_weakness_name)
    ht_o, vh_o = _ht_vh(feat_other, Wn_other, bn_other)

    q_w2v = _q(feat_weakness_name, W_w2v, 0.6)
    q_o2v = _q(feat_other, W_o2v, 0.4)
    q_v2w = _q(feat_vul, W_v2w, 1.0)
    q_v2o = _q(feat_vul, W_v2o, 1.0)

    vh_vul_p = _pack_vh(_pad_rows(vh_vul, nv + 8))
    vh_w_p = _pack_vh(_pad_rows(vh_w, nw + 8))
    vh_o_p = _pack_vh(_pad_rows(vh_o, no + 8))

    mv0, mv1 = _sc_pass(
        q_w2v, vh_vul_p, _prep_idx(src_w2v, 0), _prep_idx(dst_w2v, nv),
        q_o2v, vh_vul_p, _prep_idx(src_o2v, 0), _prep_idx(dst_o2v, nv),
        nv)
    mw, mo = _sc_pass(
        q_v2w, vh_w_p, _prep_idx(src_v2w, 0), _prep_idx(dst_v2w, nw),
        q_v2o, vh_o_p, _prep_idx(src_v2o, 0), _prep_idx(dst_v2o, no),
        nw)

    out_vul = _cat_add(ht_vul, mv0, mv1)
    out_w = _cat_add(ht_w, mw)
    out_o = _cat_add(ht_o, mo)
    return (out_vul, out_w, out_o)


# speedup vs baseline: 1.4578x; 1.0093x over previous
import functools

import jax
import jax.numpy as jnp
from jax import lax
from jax.experimental import pallas as pl
from jax.experimental.pallas import tpu as pltpu
from jax.experimental.pallas import tpu_sc as plsc

D = 256
EPS = 1e-8
EP = 40960
NTILES = 16
CH = 32
NCHUNK = EP // (NTILES * CH)


def _node_body(x_ref, wn_ref, b_ref, *rest, cs):
    nq = len(cs)
    w_refs = rest[:nq]
    ht_ref, vhp_ref = rest[nq], rest[nq + 1]
    q_refs = rest[nq + 2:]
    xb = x_ref[...]
    ht = lax.dot_general(xb, wn_ref[...], (((1,), (1,)), ((), ())),
                         preferred_element_type=jnp.float32) + b_ref[...]
    n = jnp.maximum(jnp.sqrt(jnp.sum(ht * ht, axis=1, keepdims=True)), EPS)
    ht_ref[...] = ht
    vh = ht / n
    u = lax.bitcast_convert_type(vh.astype(jnp.bfloat16), jnp.uint16)
    w32 = (u[:, :D // 2].astype(jnp.uint32)
           | (u[:, D // 2:].astype(jnp.uint32) << 16))
    vhp_ref[...] = lax.bitcast_convert_type(w32, jnp.int32)
    for wr, qr, c in zip(w_refs, q_refs, cs):
        q0 = lax.dot_general(xb, wr[...], (((1,), (1,)), ((), ())),
                             preferred_element_type=jnp.float32)
        nq_ = jnp.maximum(jnp.sqrt(jnp.sum(q0 * q0, axis=1, keepdims=True)),
                          EPS)
        qr[...] = q0 * jnp.sqrt(c / nq_)


def _node(x, wn, b, ws, cs, bm=1000):
    m = x.shape[0]
    return pl.pallas_call(
        functools.partial(_node_body, cs=tuple(cs)),
        grid=(m // bm,),
        in_specs=([pl.BlockSpec((bm, D), lambda i: (i, 0)),
                   pl.BlockSpec((D, D), lambda i: (0, 0)),
                   pl.BlockSpec((1, D), lambda i: (0, 0))]
                  + [pl.BlockSpec((D, D), lambda i: (0, 0))] * len(ws)),
        out_specs=([pl.BlockSpec((bm, D), lambda i: (i, 0)),
                    pl.BlockSpec((bm, D // 2), lambda i: (i, 0))]
                   + [pl.BlockSpec((bm, D), lambda i: (i, 0))] * len(ws)),
        out_shape=([jax.ShapeDtypeStruct((m, D), jnp.float32),
                    jax.ShapeDtypeStruct((m, D // 2), jnp.int32)]
                   + [jax.ShapeDtypeStruct((m, D), jnp.float32)] * len(ws)),
    )(x, wn, b.reshape(1, D), *ws)


def _cat_body2(ht_ref, a_ref, o_ref):
    o_ref[:, :D] = ht_ref[...]
    o_ref[:, D:] = a_ref[...]


def _cat_body3(ht_ref, a_ref, b_ref, o_ref):
    o_ref[:, :D] = ht_ref[...]
    o_ref[:, D:] = a_ref[...] + b_ref[...]


def _cat_add(ht, a, b=None, bm=1000):
    m = ht.shape[0]
    args = (ht, a) if b is None else (ht, a, b)
    return pl.pallas_call(
        _cat_body2 if b is None else _cat_body3,
        grid=(m // bm,),
        in_specs=[pl.BlockSpec((bm, D), lambda i: (i, 0))] * len(args),
        out_specs=pl.BlockSpec((bm, 2 * D), lambda i: (i, 0)),
        out_shape=jax.ShapeDtypeStruct((m, 2 * D), jnp.float32),
    )(*args)


NBUF = 4


def _edge_kernel(q0_tab, vh0_tab, q1_tab, vh1_tab, idx4_h,
                 zrows,
                 m0_ref, m1_ref,
                 acc, sidx, didx, *bufs_and_sems,
                 **kw):
    acc_rows = kw["acc_rows"]
    n_out = kw["n_out"]
    qrows = list(bufs_and_sems[0:NBUF])
    vrows = list(bufs_and_sems[NBUF:2 * NBUF])
    semq = list(bufs_and_sems[2 * NBUF:3 * NBUF])
    semv = list(bufs_and_sems[3 * NBUF:4 * NBUF])
    cid = lax.axis_index("c")
    sid = lax.axis_index("s")
    qrows0 = qrows[0]

    zcopies = acc_rows // (NTILES * CH)
    pltpu.sync_copy(zrows, qrows0)
    for k in range(zcopies):
        pltpu.sync_copy(qrows0, acc.at[pl.ds(sid * (zcopies * CH) + k * CH,
                                             CH)])
    plsc.subcore_barrier()

    def do_rel(q_tab, vh_tab, ks, kd):
        pltpu.sync_copy(idx4_h.at[ks, sid], sidx)
        pltpu.sync_copy(idx4_h.at[kd, sid], didx)

        def fire(jj, b):
            pltpu.async_copy(q_tab.at[sidx.at[jj]], qrows[b], semq[b])
            pltpu.async_copy(vh_tab.at[didx.at[jj]], vrows[b], semv[b])

        def drain_compute_scatter(jj, b):
            pltpu.make_async_copy(q_tab.at[sidx.at[jj]], qrows[b],
                                  semq[b]).wait()
            pltpu.make_async_copy(vh_tab.at[didx.at[jj]], vrows[b],
                                  semv[b]).wait()
            hi_mask = jnp.full((16,), -65536, jnp.int32)

            @plsc.parallel_loop(0, CH, step=1, unroll=2)
            def _(e):
                qs = [qrows[b][e, pl.ds(k * 16, 16)] for k in range(D // 16)]
                t = None
                for j in range(D // 32):
                    vw = vrows[b][e, pl.ds(j * 16, 16)]
                    vlo = plsc.bitcast(vw << 16, jnp.float32)
                    vhi = plsc.bitcast(vw & hi_mask, jnp.float32)
                    p = qs[j] * vlo + qs[j + 8] * vhi
                    t = p if t is None else t + p
                s = jnp.sum(t)
                for k in range(D // 16):
                    qrows[b][e, pl.ds(k * 16, 16)] = qs[k] * s

            pltpu.sync_copy(qrows[b], acc.at[didx.at[jj]], add=True)

        for b in range(NBUF):
            fire(jnp.int32(b), b)

        def chunkn(j, carry):
            for b in range(NBUF):
                jj = j + b
                drain_compute_scatter(jj, b)

                @pl.when(jj + NBUF < NCHUNK)
                def _():
                    fire(jj + NBUF, b)
            return carry

        lax.fori_loop(0, NCHUNK // NBUF, lambda i, c: chunkn(i * NBUF, c),
                      jnp.int32(0))

    @pl.when(cid == 0)
    def _():
        do_rel(q0_tab, vh0_tab, 0, 1)

    @pl.when(cid == 1)
    def _():
        do_rel(q1_tab, vh1_tab, 2, 3)

    plsc.subcore_barrier()

    main = (n_out // (NTILES * 8)) * 8
    last = n_out - 15 * main
    out_ref = [m0_ref, m1_ref]
    for c in range(2):
        @pl.when(cid == c)
        def _(oref=out_ref[c]):
            @pl.when(sid < 15)
            def _():
                pltpu.sync_copy(acc.at[pl.ds(sid * main, main)],
                                oref.at[pl.ds(sid * main, main)])

            @pl.when(sid == 15)
            def _():
                pltpu.sync_copy(acc.at[pl.ds(15 * main, last)],
                                oref.at[pl.ds(15 * main, last)])


def _sc_pass(q0, vh0, q1, vh1, idx4, n_out):
    acc_rows = ((n_out + NTILES * CH) // (NTILES * CH)) * (NTILES * CH)
    mesh = plsc.VectorSubcoreMesh(core_axis_name="c", subcore_axis_name="s")
    f = pl.kernel(
        functools.partial(_edge_kernel, acc_rows=acc_rows, n_out=n_out),
        mesh=mesh,
        compiler_params=pltpu.CompilerParams(use_tc_tiling_on_sc=False,
                                             needs_layout_passes=False),
        out_type=[jax.ShapeDtypeStruct((n_out, D), jnp.float32),
                  jax.ShapeDtypeStruct((n_out, D), jnp.float32)],
        scratch_types=([
            pltpu.VMEM_SHARED((acc_rows, D), jnp.float32),
            pltpu.VMEM((NCHUNK, CH), jnp.int32),
            pltpu.VMEM((NCHUNK, CH), jnp.int32),
        ] + [pltpu.VMEM((CH, D), jnp.float32)] * NBUF
            + [pltpu.VMEM((CH, D // 2), jnp.int32)] * NBUF
            + [pltpu.SemaphoreType.DMA] * (2 * NBUF)),
    )
    zrows = jnp.zeros((CH, D), jnp.float32)
    return f(q0, vh0, q1, vh1, idx4, zrows)


def _prep_idx4(specs):
    rows = []
    for idx, pad_val in specs:
        e = idx.shape[0]
        rows.append(jnp.concatenate([idx.astype(jnp.int32),
                                     jnp.full((EP - e,), pad_val,
                                              jnp.int32)]))
    return jnp.stack(rows).reshape(4, NTILES, NCHUNK, CH)


def _pad_rows(x, n):
    return jnp.zeros((n, x.shape[1]), x.dtype).at[:x.shape[0]].set(x)


def kernel(feat_vul, feat_weakness_name, feat_other, src_w2v, dst_w2v,
           src_o2v, dst_o2v, src_v2w, dst_v2w, src_v2o, dst_v2o,
           W_w2v, W_o2v, W_v2w, W_v2o,
           Wn_vul, bn_vul, Wn_weakness_name, bn_weakness_name,
           Wn_other, bn_other):
    nv, nw, no = feat_vul.shape[0], feat_weakness_name.shape[0], feat_other.shape[0]
    ht_vul, vhp_vul, q_v2w, q_v2o = _node(
        feat_vul, Wn_vul, bn_vul, [W_v2w, W_v2o], [1.0, 1.0])
    ht_w, vhp_w, q_w2v = _node(
        feat_weakness_name, Wn_weakness_name, bn_weakness_name,
        [W_w2v], [0.6])
    ht_o, vhp_o, q_o2v = _node(feat_other, Wn_other, bn_other,
                               [W_o2v], [0.4])

    vh_vul_p = _pad_rows(vhp_vul, nv + 8)
    vh_w_p = _pad_rows(vhp_w, nw + 8)
    vh_o_p = _pad_rows(vhp_o, no + 8)

    idx_a = _prep_idx4([(src_w2v, 0), (dst_w2v, nv),
                        (src_o2v, 0), (dst_o2v, nv)])
    mv0, mv1 = _sc_pass(q_w2v, vh_vul_p, q_o2v, vh_vul_p, idx_a, nv)
    idx_b = _prep_idx4([(src_v2w, 0), (dst_v2w, nw),
                        (src_v2o, 0), (dst_v2o, no)])
    mw, mo = _sc_pass(q_v2w, vh_w_p, q_v2o, vh_o_p, idx_b, nw)

    out_vul = _cat_add(ht_vul, mv0, mv1)
    out_w = _cat_add(ht_w, mw)
    out_o = _cat_add(ht_o, mo)
    return (out_vul, out_w, out_o)
